# baseline (speedup 1.0000x reference)
import functools

import jax
import jax.numpy as jnp
import numpy as np
from jax.experimental import pallas as pl
from jax.experimental.pallas import tpu as pltpu


def _log_sig(x):
    return jnp.minimum(x, 0.0) - jnp.log(1.0 + jnp.exp(-jnp.abs(x)))


def _chunk_sum_mat(d, chunks):
    m = np.zeros((chunks * d, chunks), np.float32)
    for k in range(chunks):
        m[k * d:(k + 1) * d, k] = 1.0
    return m


def _loss_kernel(ctr_ref, *gath_and_out, num_aspects, num_negs, bag,
                 inv_total):
    gath_refs, out_ref = gath_and_out[:-1], gath_and_out[-1]
    A, NN = num_aspects, num_negs
    f32 = jnp.float32
    ctr = ctr_ref[...]
    dots = jnp.concatenate(
        [jnp.sum(g[...] * ctr[:, None, :], axis=-1) for g in gath_refs],
        axis=1)

    sp = dots[:, :A]
    score_pos = -_log_sig(sp)
    score_neg = jnp.zeros_like(sp)
    for n in range(NN):
        score_neg = score_neg - _log_sig(-dots[:, A + n * A:A + (n + 1) * A])

    off = A + NN * A
    asp = dots[:, off:off + A]
    for j in range(1, bag):
        asp = asp + dots[:, off + j * A:off + (j + 1) * A]
    asp = asp * (1.0 / bag)

    m = jnp.max(asp, axis=-1, keepdims=True)
    e = jnp.exp(asp - m)
    w = e / jnp.sum(e, axis=-1, keepdims=True)

    tile_sum = jnp.sum(w * (score_pos + score_neg)) * inv_total
    out_ref[...] = jnp.full(out_ref.shape, tile_sum, f32)


def _reg_kernel(emb_ref, sum_a_ref, out_ref, *, num_aspects, dim, threshold,
                eps):
    A, D = num_aspects, dim
    f32 = jnp.float32
    x3 = emb_ref[...]
    x = jnp.concatenate([x3[a] for a in range(A)], axis=-1)
    sum_a = sum_a_ref[...]
    n = jnp.sqrt(jnp.dot(x * x, sum_a, preferred_element_type=f32))
    acc = jnp.zeros((), f32)
    for s in range(1, A // 2 + 1):
        r = pltpu.roll(x, s * D, axis=1)
        d = jnp.dot(x * r, sum_a, preferred_element_type=f32)
        nr = jnp.concatenate([n[:, A - s:], n[:, :A - s]], axis=1)
        sim = d / jnp.maximum(n * nr, eps)
        a = jnp.abs(sim)
        contrib = jnp.sum(jnp.where(a > threshold, a, 0.0))
        weight = 0.5 if (A % 2 == 0 and s == A // 2) else 1.0
        acc = acc + weight * contrib
    out_ref[...] = jnp.full(out_ref.shape, acc, f32)


def kernel(aspect, center, pairs, negs, offsets, lists):
    N, D = center.shape
    A = aspect.shape[0] // N
    B = pairs.shape[0]
    NN = negs.shape[1]
    L = lists.shape[0]
    bag = L // B
    K_used = A + NN * A + bag * A
    K = ((K_used + 7) // 8) * 8
    threshold, reg_coef, eps = 0.3, 0.01, 1e-8

    centers = pairs[:, 0]
    contexts = pairs[:, 1]
    aoff = (jnp.arange(A, dtype=jnp.int32) * N)

    TN = 4096 if N % 4096 == 0 else N
    GN = N // TN
    sum_a = jnp.asarray(_chunk_sum_mat(D, A))
    reg_fn = functools.partial(_reg_kernel, num_aspects=A, dim=D,
                               threshold=threshold, eps=eps)
    div_partials = pl.pallas_call(
        reg_fn,
        out_shape=jax.ShapeDtypeStruct((GN, 8, 128), jnp.float32),
        grid=(GN,),
        in_specs=[pl.BlockSpec((A, TN, D), lambda i: (0, i, 0)),
                  pl.BlockSpec((A * D, A), lambda i: (0, 0))],
        out_specs=pl.BlockSpec((1, 8, 128), lambda i: (i, 0, 0)),
        compiler_params=pltpu.CompilerParams(
            dimension_semantics=("parallel",),
            vmem_limit_bytes=48 * 1024 * 1024),
    )(aspect.reshape(A, N, D), sum_a)
    div_metric = jnp.sum(div_partials[:, 0, 0])

    npad = K // A - 1 - NN - bag
    pad_nodes = jnp.broadcast_to(contexts[:, None], (B, npad))
    nodes = jnp.concatenate(
        [contexts[:, None], negs, lists.reshape(B, bag), pad_nodes], axis=1)
    idx_all = (nodes[:, :, None] + aoff).reshape(B, K)
    gath = aspect[idx_all]
    ctr_emb = center[centers]

    TB = 256 if B % 256 == 0 else B
    G = B // TB
    SPLIT = 6 if K % 6 == 0 else 1
    KB = K // SPLIT
    loss_fn = functools.partial(_loss_kernel, num_aspects=A, num_negs=NN,
                                bag=bag, inv_total=1.0 / float(B * A))
    sg_partials = pl.pallas_call(
        loss_fn,
        out_shape=jax.ShapeDtypeStruct((G, 8, 128), jnp.float32),
        grid=(G,),
        in_specs=[pl.BlockSpec((TB, D), lambda i: (i, 0))] +
                 [pl.BlockSpec((TB, KB, D), lambda i, j=j: (i, j, 0))
                  for j in range(SPLIT)],
        out_specs=pl.BlockSpec((1, 8, 128), lambda i: (i, 0, 0)),
        compiler_params=pltpu.CompilerParams(
            dimension_semantics=("parallel",),
            vmem_limit_bytes=48 * 1024 * 1024),
    )(ctr_emb, *([gath] * SPLIT))
    sg_loss = jnp.sum(sg_partials[:, 0, 0])

    div_reg = reg_coef * div_metric
    return sg_loss + div_reg, div_reg


# speedup vs baseline: 1.0406x; 1.0406x over previous
import functools

import jax
import jax.numpy as jnp
import numpy as np
from jax.experimental import pallas as pl
from jax.experimental.pallas import tpu as pltpu


def _log_sig(x):
    return jnp.minimum(x, 0.0) - jnp.log(1.0 + jnp.exp(-jnp.abs(x)))


def _chunk_sum_mat(d, chunks):
    m = np.zeros((chunks * d, chunks), np.float32)
    for k in range(chunks):
        m[k * d:(k + 1) * d, k] = 1.0
    return m


def _loss_kernel(ctr_ref, *gath_and_out, num_aspects, num_negs, bag,
                 inv_total):
    gath_refs, out_ref = gath_and_out[:-1], gath_and_out[-1]
    A, NN = num_aspects, num_negs
    f32 = jnp.float32
    ctr = ctr_ref[...][None, :, :]
    dots = jnp.concatenate(
        [jnp.sum(g[...] * ctr, axis=-1) for g in gath_refs],
        axis=0)

    score = [None] * A
    for a in range(A):
        s = -_log_sig(dots[a])
        for n in range(NN):
            s = s - _log_sig(-dots[A + n * A + a])
        score[a] = s

    off = A + NN * A
    asp = []
    for a in range(A):
        t = dots[off + a]
        for j in range(1, bag):
            t = t + dots[off + j * A + a]
        asp.append(t * (1.0 / bag))

    m = jnp.maximum(jnp.maximum(asp[0], asp[1]), jnp.maximum(asp[2], asp[3])) \
        if A == 4 else functools.reduce(jnp.maximum, asp)
    e = [jnp.exp(t - m) for t in asp]
    inv = 1.0 / functools.reduce(jnp.add, e)
    tile_sum = functools.reduce(
        jnp.add, [jnp.sum(e[a] * inv * score[a]) for a in range(A)])
    out_ref[...] = jnp.full(out_ref.shape, tile_sum * inv_total, f32)


def _reg_kernel(emb_ref, sum_a_ref, out_ref, *, num_aspects, dim, threshold,
                eps):
    A, D = num_aspects, dim
    f32 = jnp.float32
    x3 = emb_ref[...]
    x = jnp.concatenate([x3[a] for a in range(A)], axis=-1)
    sum_a = sum_a_ref[...]
    n = jnp.sqrt(jnp.dot(x * x, sum_a, preferred_element_type=f32))
    acc = jnp.zeros((), f32)
    for s in range(1, A // 2 + 1):
        r = pltpu.roll(x, s * D, axis=1)
        d = jnp.dot(x * r, sum_a, preferred_element_type=f32)
        nr = jnp.concatenate([n[:, A - s:], n[:, :A - s]], axis=1)
        sim = d / jnp.maximum(n * nr, eps)
        a = jnp.abs(sim)
        contrib = jnp.sum(jnp.where(a > threshold, a, 0.0))
        weight = 0.5 if (A % 2 == 0 and s == A // 2) else 1.0
        acc = acc + weight * contrib
    out_ref[...] = jnp.full(out_ref.shape, acc, f32)


def kernel(aspect, center, pairs, negs, offsets, lists):
    N, D = center.shape
    A = aspect.shape[0] // N
    B = pairs.shape[0]
    NN = negs.shape[1]
    L = lists.shape[0]
    bag = L // B
    K_used = A + NN * A + bag * A
    K = ((K_used + 7) // 8) * 8
    threshold, reg_coef, eps = 0.3, 0.01, 1e-8

    centers = pairs[:, 0]
    contexts = pairs[:, 1]
    aoff = (jnp.arange(A, dtype=jnp.int32) * N)

    TN = 4096 if N % 4096 == 0 else N
    GN = N // TN
    sum_a = jnp.asarray(_chunk_sum_mat(D, A))
    reg_fn = functools.partial(_reg_kernel, num_aspects=A, dim=D,
                               threshold=threshold, eps=eps)
    div_partials = pl.pallas_call(
        reg_fn,
        out_shape=jax.ShapeDtypeStruct((GN, 8, 128), jnp.float32),
        grid=(GN,),
        in_specs=[pl.BlockSpec((A, TN, D), lambda i: (0, i, 0)),
                  pl.BlockSpec((A * D, A), lambda i: (0, 0))],
        out_specs=pl.BlockSpec((1, 8, 128), lambda i: (i, 0, 0)),
        compiler_params=pltpu.CompilerParams(
            dimension_semantics=("parallel",),
            vmem_limit_bytes=48 * 1024 * 1024),
    )(aspect.reshape(A, N, D), sum_a)
    div_metric = jnp.sum(div_partials[:, 0, 0])

    npad = K // A - 1 - NN - bag
    pad_nodes = jnp.broadcast_to(contexts[None, :], (npad, B))
    nodes = jnp.concatenate(
        [contexts[None, :], negs.T, lists.reshape(B, bag).T, pad_nodes],
        axis=0)
    idx_all = (nodes[:, None, :] + aoff[None, :, None]).reshape(K, B)
    gath = aspect[idx_all]
    ctr_emb = center[centers]

    TB = 256 if B % 256 == 0 else B
    G = B // TB
    SPLIT = 6 if K % 6 == 0 else 1
    KB = K // SPLIT
    loss_fn = functools.partial(_loss_kernel, num_aspects=A, num_negs=NN,
                                bag=bag, inv_total=1.0 / float(B * A))
    sg_partials = pl.pallas_call(
        loss_fn,
        out_shape=jax.ShapeDtypeStruct((G, 8, 128), jnp.float32),
        grid=(G,),
        in_specs=[pl.BlockSpec((TB, D), lambda i: (i, 0))] +
                 [pl.BlockSpec((KB, TB, D), lambda i, j=j: (j, i, 0))
                  for j in range(SPLIT)],
        out_specs=pl.BlockSpec((1, 8, 128), lambda i: (i, 0, 0)),
        compiler_params=pltpu.CompilerParams(
            dimension_semantics=("parallel",),
            vmem_limit_bytes=48 * 1024 * 1024),
    )(ctr_emb, *([gath] * SPLIT))
    sg_loss = jnp.sum(sg_partials[:, 0, 0])

    div_reg = reg_coef * div_metric
    return sg_loss + div_reg, div_reg


# speedup vs baseline: 1.0420x; 1.0013x over previous
import functools

import jax
import jax.numpy as jnp
import numpy as np
from jax.experimental import pallas as pl
from jax.experimental.pallas import tpu as pltpu


def _log_sig(x):
    return jnp.minimum(x, 0.0) - jnp.log(1.0 + jnp.exp(-jnp.abs(x)))


def _chunk_sum_mat(d, chunks):
    m = np.zeros((chunks * d, chunks), np.float32)
    for k in range(chunks):
        m[k * d:(k + 1) * d, k] = 1.0
    return m


def _loss_kernel(ctr_ref, *gath_and_out, num_aspects, num_negs, bag,
                 inv_total):
    gath_refs, out_ref = gath_and_out[:-1], gath_and_out[-1]
    A, NN = num_aspects, num_negs
    f32 = jnp.float32
    ctr = ctr_ref[...][None, :, :]
    dots = jnp.concatenate(
        [jnp.sum(g[...] * ctr, axis=-1) for g in gath_refs],
        axis=0)

    score = [None] * A
    for a in range(A):
        s = -_log_sig(dots[a])
        for n in range(NN):
            s = s - _log_sig(-dots[A + n * A + a])
        score[a] = s

    off = A + NN * A
    asp = []
    for a in range(A):
        t = dots[off + a]
        for j in range(1, bag):
            t = t + dots[off + j * A + a]
        asp.append(t * (1.0 / bag))

    m = jnp.maximum(jnp.maximum(asp[0], asp[1]), jnp.maximum(asp[2], asp[3])) \
        if A == 4 else functools.reduce(jnp.maximum, asp)
    e = [jnp.exp(t - m) for t in asp]
    inv = 1.0 / functools.reduce(jnp.add, e)
    tile_sum = functools.reduce(
        jnp.add, [jnp.sum(e[a] * inv * score[a]) for a in range(A)])
    out_ref[...] = jnp.full(out_ref.shape, tile_sum * inv_total, f32)


def _reg_kernel(*refs, num_aspects, dim, threshold, eps):
    A, D = num_aspects, dim
    emb_refs, sum_a_ref, out_ref = refs[:A], refs[A], refs[A + 1]
    f32 = jnp.float32
    x = jnp.concatenate([emb_refs[a][0] for a in range(A)], axis=-1)
    sum_a = sum_a_ref[...]
    n = jnp.sqrt(jnp.dot(x * x, sum_a, preferred_element_type=f32))
    acc = jnp.zeros((), f32)
    for s in range(1, A // 2 + 1):
        r = pltpu.roll(x, s * D, axis=1)
        d = jnp.dot(x * r, sum_a, preferred_element_type=f32)
        nr = jnp.concatenate([n[:, A - s:], n[:, :A - s]], axis=1)
        sim = d / jnp.maximum(n * nr, eps)
        a = jnp.abs(sim)
        contrib = jnp.sum(jnp.where(a > threshold, a, 0.0))
        weight = 0.5 if (A % 2 == 0 and s == A // 2) else 1.0
        acc = acc + weight * contrib
    out_ref[...] = jnp.full(out_ref.shape, acc, f32)


def kernel(aspect, center, pairs, negs, offsets, lists):
    N, D = center.shape
    A = aspect.shape[0] // N
    B = pairs.shape[0]
    NN = negs.shape[1]
    L = lists.shape[0]
    bag = L // B
    K_used = A + NN * A + bag * A
    K = ((K_used + 7) // 8) * 8
    threshold, reg_coef, eps = 0.3, 0.01, 1e-8

    centers = pairs[:, 0]
    contexts = pairs[:, 1]
    aoff = (jnp.arange(A, dtype=jnp.int32) * N)

    TN = 4096 if N % 4096 == 0 else N
    GN = N // TN
    sum_a = jnp.asarray(_chunk_sum_mat(D, A))
    reg_fn = functools.partial(_reg_kernel, num_aspects=A, dim=D,
                               threshold=threshold, eps=eps)
    div_partials = pl.pallas_call(
        reg_fn,
        out_shape=jax.ShapeDtypeStruct((GN, 8, 128), jnp.float32),
        grid=(GN,),
        in_specs=[pl.BlockSpec((1, TN, D), lambda i, a=a: (a, i, 0))
                  for a in range(A)] +
                 [pl.BlockSpec((A * D, A), lambda i: (0, 0))],
        out_specs=pl.BlockSpec((1, 8, 128), lambda i: (i, 0, 0)),
        compiler_params=pltpu.CompilerParams(
            dimension_semantics=("parallel",),
            vmem_limit_bytes=48 * 1024 * 1024),
    )(*([aspect.reshape(A, N, D)] * A), sum_a)
    div_metric = jnp.sum(div_partials[:, 0, 0])

    npad = K // A - 1 - NN - bag
    pad_nodes = jnp.broadcast_to(contexts[None, :], (npad, B))
    nodes = jnp.concatenate(
        [contexts[None, :], negs.T, lists.reshape(B, bag).T, pad_nodes],
        axis=0)
    idx_all = (nodes[:, None, :] + aoff[None, :, None]).reshape(K, B)
    gath = aspect[idx_all]
    ctr_emb = center[centers]

    TB = 256 if B % 256 == 0 else B
    G = B // TB
    SPLIT = 6 if K % 6 == 0 else 1
    KB = K // SPLIT
    loss_fn = functools.partial(_loss_kernel, num_aspects=A, num_negs=NN,
                                bag=bag, inv_total=1.0 / float(B * A))
    sg_partials = pl.pallas_call(
        loss_fn,
        out_shape=jax.ShapeDtypeStruct((G, 8, 128), jnp.float32),
        grid=(G,),
        in_specs=[pl.BlockSpec((TB, D), lambda i: (i, 0))] +
                 [pl.BlockSpec((KB, TB, D), lambda i, j=j: (j, i, 0))
                  for j in range(SPLIT)],
        out_specs=pl.BlockSpec((1, 8, 128), lambda i: (i, 0, 0)),
        compiler_params=pltpu.CompilerParams(
            dimension_semantics=("parallel",),
            vmem_limit_bytes=48 * 1024 * 1024),
    )(ctr_emb, *([gath] * SPLIT))
    sg_loss = jnp.sum(sg_partials[:, 0, 0])

    div_reg = reg_coef * div_metric
    return sg_loss + div_reg, div_reg


# speedup vs baseline: 1.0487x; 1.0064x over previous
import functools

import jax
import jax.numpy as jnp
import numpy as np
from jax.experimental import pallas as pl
from jax.experimental.pallas import tpu as pltpu


def _log_sig(x):
    return jnp.minimum(x, 0.0) - jnp.log(1.0 + jnp.exp(-jnp.abs(x)))


def _chunk_sum_mat(d, chunks):
    m = np.zeros((chunks * d, chunks), np.float32)
    for k in range(chunks):
        m[k * d:(k + 1) * d, k] = 1.0
    return m


def _loss_kernel(ctr_ref, *gath_and_out, num_aspects, num_negs, bag,
                 inv_total):
    gath_refs, out_ref = gath_and_out[:-1], gath_and_out[-1]
    A, NN = num_aspects, num_negs
    f32 = jnp.float32
    ctr = ctr_ref[...][None, :, :]
    dots = jnp.concatenate(
        [jnp.sum(g[...] * ctr, axis=-1) for g in gath_refs],
        axis=0)

    score = [None] * A
    for a in range(A):
        s = -_log_sig(dots[a])
        for n in range(NN):
            s = s - _log_sig(-dots[A + n * A + a])
        score[a] = s

    off = A + NN * A
    asp = []
    for a in range(A):
        t = dots[off + a]
        for j in range(1, bag):
            t = t + dots[off + j * A + a]
        asp.append(t * (1.0 / bag))

    m = jnp.maximum(jnp.maximum(asp[0], asp[1]), jnp.maximum(asp[2], asp[3])) \
        if A == 4 else functools.reduce(jnp.maximum, asp)
    e = [jnp.exp(t - m) for t in asp]
    inv = 1.0 / functools.reduce(jnp.add, e)
    tile_sum = functools.reduce(
        jnp.add, [jnp.sum(e[a] * inv * score[a]) for a in range(A)])
    out_ref[...] = jnp.full(out_ref.shape, tile_sum * inv_total, f32)


def _reg_kernel(*refs, num_aspects, dim, threshold, eps):
    A, D = num_aspects, dim
    emb_refs, sum_a_ref, out_ref = refs[:A], refs[A], refs[A + 1]
    f32 = jnp.float32
    x = jnp.concatenate([emb_refs[a][0] for a in range(A)], axis=-1)
    sum_a = sum_a_ref[...]
    n = jnp.sqrt(jnp.dot(x * x, sum_a, preferred_element_type=f32))
    acc = jnp.zeros((), f32)
    for s in range(1, A // 2 + 1):
        r = pltpu.roll(x, s * D, axis=1)
        d = jnp.dot(x * r, sum_a, preferred_element_type=f32)
        nr = jnp.concatenate([n[:, A - s:], n[:, :A - s]], axis=1)
        sim = d / jnp.maximum(n * nr, eps)
        a = jnp.abs(sim)
        contrib = jnp.sum(jnp.where(a > threshold, a, 0.0))
        weight = 0.5 if (A % 2 == 0 and s == A // 2) else 1.0
        acc = acc + weight * contrib
    out_ref[...] = jnp.full(out_ref.shape, acc, f32)


def kernel(aspect, center, pairs, negs, offsets, lists):
    N, D = center.shape
    A = aspect.shape[0] // N
    B = pairs.shape[0]
    NN = negs.shape[1]
    L = lists.shape[0]
    bag = L // B
    K_used = A + NN * A + bag * A
    K = ((K_used + 7) // 8) * 8
    threshold, reg_coef, eps = 0.3, 0.01, 1e-8

    centers = pairs[:, 0]
    contexts = pairs[:, 1]
    aoff = (jnp.arange(A, dtype=jnp.int32) * N)

    TN = 4096 if N % 4096 == 0 else N
    GN = N // TN
    sum_a = jnp.asarray(_chunk_sum_mat(D, A))
    reg_fn = functools.partial(_reg_kernel, num_aspects=A, dim=D,
                               threshold=threshold, eps=eps)
    div_partials = pl.pallas_call(
        reg_fn,
        out_shape=jax.ShapeDtypeStruct((GN, 8, 128), jnp.float32),
        grid=(GN,),
        in_specs=[pl.BlockSpec((1, TN, D), lambda i, a=a: (a, i, 0))
                  for a in range(A)] +
                 [pl.BlockSpec((A * D, A), lambda i: (0, 0))],
        out_specs=pl.BlockSpec((1, 8, 128), lambda i: (i, 0, 0)),
        compiler_params=pltpu.CompilerParams(
            dimension_semantics=("parallel",),
            vmem_limit_bytes=48 * 1024 * 1024),
    )(*([aspect.reshape(A, N, D)] * A), sum_a)
    div_metric = jnp.sum(div_partials[:, 0, 0])

    npad = K // A - 1 - NN - bag
    pad_nodes = jnp.broadcast_to(contexts[None, :], (npad, B))
    nodes = jnp.concatenate(
        [contexts[None, :], negs.T, lists.reshape(B, bag).T, pad_nodes],
        axis=0)
    idx_all = (nodes[:, None, :] + aoff[None, :, None]).reshape(K, B)
    gath = aspect[idx_all]
    ctr_emb = center[centers]

    TB = 512 if B % 512 == 0 else B
    G = B // TB
    SPLIT = 6 if K % 6 == 0 else 1
    KB = K // SPLIT
    loss_fn = functools.partial(_loss_kernel, num_aspects=A, num_negs=NN,
                                bag=bag, inv_total=1.0 / float(B * A))
    sg_partials = pl.pallas_call(
        loss_fn,
        out_shape=jax.ShapeDtypeStruct((G, 8, 128), jnp.float32),
        grid=(G,),
        in_specs=[pl.BlockSpec((TB, D), lambda i: (i, 0))] +
                 [pl.BlockSpec((KB, TB, D), lambda i, j=j: (j, i, 0))
                  for j in range(SPLIT)],
        out_specs=pl.BlockSpec((1, 8, 128), lambda i: (i, 0, 0)),
        compiler_params=pltpu.CompilerParams(
            dimension_semantics=("parallel",),
            vmem_limit_bytes=48 * 1024 * 1024),
    )(ctr_emb, *([gath] * SPLIT))
    sg_loss = jnp.sum(sg_partials[:, 0, 0])

    div_reg = reg_coef * div_metric
    return sg_loss + div_reg, div_reg
